# baseline (speedup 1.0000x reference)
import functools

import jax
import jax.numpy as jnp
from jax import lax
from jax.experimental import pallas as pl
from jax.experimental.pallas import tpu as pltpu
from jax.experimental.pallas import tpu_sc as plsc

N = 10000
E = 160000
D = 256
NEG = 0.01

NW = 32
L = 16
EPW = E // NW
SPW = (N + NW - 1) // NW
NPAD = 10240
HALF = N // 2
PHALF = HALF + 8
NREAL = 2 * PHALF
RPT = D // NW
RPP = RPT // 2
CH = 8000
NCHK = E // CH
NV = CH // L
NSELF = NREAL // L

_i32 = jnp.int32
_f32 = jnp.float32

_MESH = plsc.VectorSubcoreMesh(core_axis_name="c", subcore_axis_name="s")
_PARAMS = pltpu.CompilerParams(needs_layout_passes=False)


def _dis_body(degp_ref, dis_ref):
    sm = jnp.sum(degp_ref[...], axis=0, keepdims=True)
    dis_ref[...] = jnp.where(sm > 0, lax.rsqrt(jnp.maximum(sm, 1e-30)), 0.0)


def _dis_from_partials(degp):
    out = pl.pallas_call(
        _dis_body,
        out_shape=jax.ShapeDtypeStruct((1, NPAD), _f32),
    )(degp)
    return out.reshape(NPAD)


def _mmt_body(wt_ref, h_ref, dis_ref, o_ref):
    o_ref[...] = (jnp.dot(wt_ref[...], h_ref[...],
                          preferred_element_type=_f32) * dis_ref[...])


def _mmt(Wt, hT, disr):
    return pl.pallas_call(
        _mmt_body,
        out_shape=jax.ShapeDtypeStruct((D, NREAL), _f32),
    )(Wt, hT, disr)


@functools.partial(
    pl.kernel,
    out_type=(
        jax.ShapeDtypeStruct((NW, NPAD), _f32),
        jax.ShapeDtypeStruct((E,), _i32),
    ),
    mesh=_MESH,
    compiler_params=_PARAMS,
    scratch_types=(
        pltpu.VMEM((NPAD,), _f32),
        pltpu.VMEM((EPW + L,), _i32),
        pltpu.VMEM((EPW + L,), _i32),
        pltpu.VMEM((EPW + L,), _f32),
    ),
)
def _phase_a(src_ref, dst_ref, ea_ref, degp_ref, opk_ref,
             degacc, csrc, cdst, cw):
    wid = lax.axis_index("s") * 2 + lax.axis_index("c")
    lane = lax.iota(_i32, L)
    zf = jnp.zeros((L,), _f32)

    def zdeg(i, _):
        degacc[pl.ds(i * L, L)] = zf
        return 0
    lax.fori_loop(0, NPAD // L, zdeg, 0)

    base = wid * EPW
    pltpu.sync_copy(src_ref.at[pl.ds(base, EPW)], csrc.at[pl.ds(0, EPW)])
    pltpu.sync_copy(dst_ref.at[pl.ds(base, EPW)], cdst.at[pl.ds(0, EPW)])
    pltpu.sync_copy(ea_ref.at[pl.ds(base, EPW)], cw.at[pl.ds(0, EPW)])

    def edge_step(i, _):
        sl = pl.ds(i * L, L)
        srcv = csrc[sl]
        dstv = cdst[sl]
        wv = cw[sl]
        valid = (i * L + lane) < EPW
        dstv = jnp.where(valid, dstv, 0)
        pdst = dstv + 8 * (dstv >= HALF).astype(_i32)
        psrc = srcv + 8 * (srcv >= HALF).astype(_i32)
        csrc[sl] = psrc | (pdst << 16)
        plsc.addupdate_scatter(degacc, [pdst], wv, mask=valid)
        return 0
    lax.fori_loop(0, (EPW + L - 1) // L, edge_step, 0)
    pltpu.sync_copy(csrc.at[pl.ds(0, EPW)], opk_ref.at[pl.ds(base, EPW)])

    sbase = wid * SPW

    def self_step(i, _):
        node = sbase + i * L + lane
        valid = ((i * L + lane) < SPW) & (node < N)
        node = jnp.where(valid, node, 0)
        pnode = node + 8 * (node >= HALF).astype(_i32)
        ones = jnp.where(valid, 1.0, 0.0)
        plsc.addupdate_scatter(degacc, [pnode], ones, mask=valid)
        return 0
    lax.fori_loop(0, (SPW + L - 1) // L, self_step, 0)

    pltpu.sync_copy(degacc, degp_ref.at[wid])


@functools.partial(
    pl.kernel,
    out_type=jax.ShapeDtypeStruct((D, NREAL), _f32),
    mesh=_MESH,
    compiler_params=_PARAMS,
    scratch_types=(
        pltpu.VMEM((RPP, NREAL), _f32),
        pltpu.VMEM((RPP, NREAL), _f32),
        pltpu.VMEM((NPAD,), _f32),
        pltpu.VMEM((D,), _f32),
        pltpu.VMEM((CH,), _i32),
        pltpu.VMEM((CH,), _f32),
        pltpu.VMEM((CH,), _i32),
        pltpu.VMEM((CH,), _f32),
        pltpu.SemaphoreType.DMA,
        pltpu.SemaphoreType.DMA,
    ),
)
def _phase_b(y_ref, dis_ref, b_ref, pk_ref, ea_ref, out_ref,
             myh, acc, disv, bv, cpk, cwb, cpk2, cwb2, semA, semB):
    wid = lax.axis_index("s") * 2 + lax.axis_index("c")
    lane = lax.iota(_i32, L)
    zf = jnp.zeros((L,), _f32)

    pltpu.sync_copy(dis_ref, disv)
    pltpu.sync_copy(b_ref, bv)

    for p in range(2):
        row0 = wid * RPT + p * RPP
        pltpu.sync_copy(y_ref.at[pl.ds(row0, RPP)], myh)

        @plsc.parallel_loop(0, NSELF, step=1, unroll=8)
        def _(i):
            sl = pl.ds(i * L, L)
            for r in range(RPP):
                acc[r, sl] = zf

        def process(pkb, wb):
            @plsc.parallel_loop(0, NV, step=1, unroll=32)
            def _(i):
                sl = pl.ds(i * L, L)
                pk = pkb[sl]
                psrc = pk & 0xFFFF
                pdst = lax.shift_right_logical(pk, 16)
                wv = wb[sl]
                vals = [plsc.load_gather(myh, [jnp.full((L,), r, _i32), psrc])
                        * wv for r in range(RPP)]
                for r in range(RPP):
                    plsc.addupdate_scatter(
                        acc, [jnp.full((L,), r, _i32), pdst], vals[r])

        def start(jc, pkb, wb, sem):
            pltpu.async_copy(pk_ref.at[pl.ds(jc * CH, CH)], pkb, sem)
            pltpu.async_copy(ea_ref.at[pl.ds(jc * CH, CH)], wb, sem)

        def drain(pkb, wb, sem):
            pltpu.make_async_copy(pk_ref.at[pl.ds(0, CH)], pkb, sem).wait()
            pltpu.make_async_copy(ea_ref.at[pl.ds(0, CH)], wb, sem).wait()

        start(0, cpk, cwb, semA)

        def pair(jp, _):
            a = 2 * jp
            start(a + 1, cpk2, cwb2, semB)
            drain(cpk, cwb, semA)
            process(cpk, cwb)

            @pl.when(a + 2 < NCHK)
            def _():
                start(a + 2, cpk, cwb, semA)

            drain(cpk2, cwb2, semB)
            process(cpk2, cwb2)
            return 0
        lax.fori_loop(0, NCHK // 2, pair, 0)

        for r in range(RPP):
            brow = plsc.load_gather(bv, [jnp.full((L,), row0 + r, _i32)])

            @plsc.parallel_loop(0, NSELF, step=1, unroll=8)
            def _(i):
                sl = pl.ds(i * L, L)
                o = (acc[r, sl] + myh[r, sl]) * disv[sl] + brow
                acc[r, sl] = jnp.where(o > 0, o, NEG * o)
            pltpu.sync_copy(acc.at[r], out_ref.at[row0 + r])


def kernel(x, edge_index, edge_attr, W1, b1, W2, b2):
    src = edge_index[0]
    dst = edge_index[1]
    degp, pk = _phase_a(src, dst, edge_attr)
    dis = _dis_from_partials(degp)
    disr = dis[:NREAL].reshape(1, NREAL)
    zpad = jnp.zeros((8, D), _f32)
    xT = jnp.concatenate([x[:HALF], zpad, x[HALF:], zpad]).T
    y1 = _mmt(W1.T, xT, disr)
    h1 = _phase_b(y1, dis, b1, pk, edge_attr)
    y2 = _mmt(W2.T, h1, disr)
    h2 = _phase_b(y2, dis, b2, pk, edge_attr)
    h = h2.T
    return jnp.concatenate([h[:HALF], h[PHALF:PHALF + HALF]])


# speedup vs baseline: 1.0283x; 1.0283x over previous
import functools

import jax
import jax.numpy as jnp
from jax import lax
from jax.experimental import pallas as pl
from jax.experimental.pallas import tpu as pltpu
from jax.experimental.pallas import tpu_sc as plsc

N = 10000
E = 160000
D = 256
NEG = 0.01

NW = 32
L = 16
EPW = E // NW
SPW = (N + NW - 1) // NW
NPAD = 10240
HALF = N // 2
PHALF = HALF + 8
NREAL = 2 * PHALF
RPT = D // NW
RPP = RPT // 2
CH = 8000
NCHK = E // CH
NV = CH // L
NSELF = NREAL // L

_i32 = jnp.int32
_f32 = jnp.float32

_MESH = plsc.VectorSubcoreMesh(core_axis_name="c", subcore_axis_name="s")
_PARAMS = pltpu.CompilerParams(needs_layout_passes=False)


def _mmt1_body(wt_ref, h_ref, degp_ref, o_ref, dis_ref):
    sm = jnp.sum(degp_ref[...], axis=0, keepdims=True)
    dis = jnp.where(sm > 0, lax.rsqrt(jnp.maximum(sm, 1e-30)), 0.0)
    dis_ref[...] = dis
    o_ref[...] = (jnp.dot(wt_ref[...], h_ref[...],
                          preferred_element_type=_f32) * dis[:, :NREAL])


def _mmt1(Wt, hT, degp):
    return pl.pallas_call(
        _mmt1_body,
        out_shape=(jax.ShapeDtypeStruct((D, NREAL), _f32),
                   jax.ShapeDtypeStruct((1, NPAD), _f32)),
    )(Wt, hT, degp)


def _mmt_body(wt_ref, h_ref, dis_ref, o_ref):
    o_ref[...] = (jnp.dot(wt_ref[...], h_ref[...],
                          preferred_element_type=_f32)
                  * dis_ref[...][:, :NREAL])


def _mmt(Wt, hT, dis2d):
    return pl.pallas_call(
        _mmt_body,
        out_shape=jax.ShapeDtypeStruct((D, NREAL), _f32),
    )(Wt, hT, dis2d)


@functools.partial(
    pl.kernel,
    out_type=(
        jax.ShapeDtypeStruct((NW, NPAD), _f32),
        jax.ShapeDtypeStruct((E,), _i32),
    ),
    mesh=_MESH,
    compiler_params=_PARAMS,
    scratch_types=(
        pltpu.VMEM((NPAD,), _f32),
        pltpu.VMEM((EPW + L,), _i32),
        pltpu.VMEM((EPW + L,), _i32),
        pltpu.VMEM((EPW + L,), _f32),
    ),
)
def _phase_a(src_ref, dst_ref, ea_ref, degp_ref, opk_ref,
             degacc, csrc, cdst, cw):
    wid = lax.axis_index("s") * 2 + lax.axis_index("c")
    lane = lax.iota(_i32, L)
    zf = jnp.zeros((L,), _f32)

    def zdeg(i, _):
        degacc[pl.ds(i * L, L)] = zf
        return 0
    lax.fori_loop(0, NPAD // L, zdeg, 0)

    base = wid * EPW
    pltpu.sync_copy(src_ref.at[pl.ds(base, EPW)], csrc.at[pl.ds(0, EPW)])
    pltpu.sync_copy(dst_ref.at[pl.ds(base, EPW)], cdst.at[pl.ds(0, EPW)])
    pltpu.sync_copy(ea_ref.at[pl.ds(base, EPW)], cw.at[pl.ds(0, EPW)])

    def edge_step(i, _):
        sl = pl.ds(i * L, L)
        srcv = csrc[sl]
        dstv = cdst[sl]
        wv = cw[sl]
        valid = (i * L + lane) < EPW
        dstv = jnp.where(valid, dstv, 0)
        pdst = dstv + 8 * (dstv >= HALF).astype(_i32)
        psrc = srcv + 8 * (srcv >= HALF).astype(_i32)
        csrc[sl] = psrc | (pdst << 16)
        plsc.addupdate_scatter(degacc, [pdst], wv, mask=valid)
        return 0
    lax.fori_loop(0, (EPW + L - 1) // L, edge_step, 0)
    pltpu.sync_copy(csrc.at[pl.ds(0, EPW)], opk_ref.at[pl.ds(base, EPW)])

    sbase = wid * SPW

    def self_step(i, _):
        node = sbase + i * L + lane
        valid = ((i * L + lane) < SPW) & (node < N)
        node = jnp.where(valid, node, 0)
        pnode = node + 8 * (node >= HALF).astype(_i32)
        ones = jnp.where(valid, 1.0, 0.0)
        plsc.addupdate_scatter(degacc, [pnode], ones, mask=valid)
        return 0
    lax.fori_loop(0, (SPW + L - 1) // L, self_step, 0)

    pltpu.sync_copy(degacc, degp_ref.at[wid])


@functools.partial(
    pl.kernel,
    out_type=jax.ShapeDtypeStruct((D, NREAL), _f32),
    mesh=_MESH,
    compiler_params=_PARAMS,
    scratch_types=(
        pltpu.VMEM((RPP, NREAL), _f32),
        pltpu.VMEM((RPP, NREAL), _f32),
        pltpu.VMEM((NPAD,), _f32),
        pltpu.VMEM((D,), _f32),
        pltpu.VMEM((CH,), _i32),
        pltpu.VMEM((CH,), _f32),
        pltpu.VMEM((CH,), _i32),
        pltpu.VMEM((CH,), _f32),
        pltpu.SemaphoreType.DMA,
        pltpu.SemaphoreType.DMA,
    ),
)
def _phase_b(y_ref, dis_ref, b_ref, pk_ref, ea_ref, out_ref,
             myh, acc, disv, bv, cpk, cwb, cpk2, cwb2, semA, semB):
    wid = lax.axis_index("s") * 2 + lax.axis_index("c")
    lane = lax.iota(_i32, L)
    zf = jnp.zeros((L,), _f32)

    pltpu.sync_copy(dis_ref, disv)
    pltpu.sync_copy(b_ref, bv)

    for p in range(2):
        row0 = wid * RPT + p * RPP
        pltpu.sync_copy(y_ref.at[pl.ds(row0, RPP)], myh)

        @plsc.parallel_loop(0, NSELF, step=1, unroll=8)
        def _(i):
            sl = pl.ds(i * L, L)
            for r in range(RPP):
                acc[r, sl] = zf

        def process(pkb, wb):
            @plsc.parallel_loop(0, NV, step=1, unroll=25)
            def _(i):
                sl = pl.ds(i * L, L)
                pk = pkb[sl]
                psrc = pk & 0xFFFF
                pdst = lax.shift_right_logical(pk, 16)
                wv = wb[sl]
                vals = [plsc.load_gather(myh, [jnp.full((L,), r, _i32), psrc])
                        * wv for r in range(RPP)]
                for r in range(RPP):
                    plsc.addupdate_scatter(
                        acc, [jnp.full((L,), r, _i32), pdst], vals[r])

        def start(jc, pkb, wb, sem):
            pltpu.async_copy(pk_ref.at[pl.ds(jc * CH, CH)], pkb, sem)
            pltpu.async_copy(ea_ref.at[pl.ds(jc * CH, CH)], wb, sem)

        def drain(pkb, wb, sem):
            pltpu.make_async_copy(pk_ref.at[pl.ds(0, CH)], pkb, sem).wait()
            pltpu.make_async_copy(ea_ref.at[pl.ds(0, CH)], wb, sem).wait()

        start(0, cpk, cwb, semA)

        def pair(jp, _):
            a = 2 * jp
            start(a + 1, cpk2, cwb2, semB)
            drain(cpk, cwb, semA)
            process(cpk, cwb)

            @pl.when(a + 2 < NCHK)
            def _():
                start(a + 2, cpk, cwb, semA)

            drain(cpk2, cwb2, semB)
            process(cpk2, cwb2)
            return 0
        lax.fori_loop(0, NCHK // 2, pair, 0)

        for r in range(RPP):
            brow = plsc.load_gather(bv, [jnp.full((L,), row0 + r, _i32)])

            @plsc.parallel_loop(0, NSELF, step=1, unroll=8)
            def _(i):
                sl = pl.ds(i * L, L)
                o = (acc[r, sl] + myh[r, sl]) * disv[sl] + brow
                acc[r, sl] = jnp.where(o > 0, o, NEG * o)
            pltpu.sync_copy(acc.at[r], out_ref.at[row0 + r])


def kernel(x, edge_index, edge_attr, W1, b1, W2, b2):
    src = edge_index[0]
    dst = edge_index[1]
    degp, pk = _phase_a(src, dst, edge_attr)
    zpad = jnp.zeros((8, D), _f32)
    xT = jnp.concatenate([x[:HALF], zpad, x[HALF:], zpad]).T
    y1, dis2d = _mmt1(W1.T, xT, degp)
    dis = dis2d.reshape(NPAD)
    h1 = _phase_b(y1, dis, b1, pk, edge_attr)
    y2 = _mmt(W2.T, h1, dis2d)
    h2 = _phase_b(y2, dis, b2, pk, edge_attr)
    h = h2.T
    return jnp.concatenate([h[:HALF], h[PHALF:PHALF + HALF]])


# speedup vs baseline: 1.0589x; 1.0298x over previous
import functools

import jax
import jax.numpy as jnp
from jax import lax
from jax.experimental import pallas as pl
from jax.experimental.pallas import tpu as pltpu
from jax.experimental.pallas import tpu_sc as plsc

N = 10000
E = 160000
D = 256
NEG = 0.01

NW = 32
L = 16
EPW = E // NW
SPW = (N + NW - 1) // NW
NPAD = 10240
HALF = N // 2
PHALF = HALF + 8
NREAL = 2 * PHALF
RPT = D // NW
RPP = RPT // 2
CH = 8000
NCHK = E // CH
NV = CH // L
NSELF = NREAL // L

_i32 = jnp.int32
_f32 = jnp.float32

_MESH = plsc.VectorSubcoreMesh(core_axis_name="c", subcore_axis_name="s")
_PARAMS = pltpu.CompilerParams(needs_layout_passes=False)


def _mmt1_body(wt_ref, x_ref, degp_ref, o_ref, dis_ref):
    sm = jnp.sum(degp_ref[...], axis=0, keepdims=True)
    dis = jnp.where(sm > 0, lax.rsqrt(jnp.maximum(sm, 1e-30)), 0.0)
    dis_ref[...] = dis
    yt = lax.dot_general(wt_ref[...], x_ref[...],
                         (((1,), (1,)), ((), ())),
                         preferred_element_type=_f32)
    o_ref[...] = yt * dis[:, :NREAL]


def _mmt1(Wt, hT, degp):
    return pl.pallas_call(
        _mmt1_body,
        out_shape=(jax.ShapeDtypeStruct((D, NREAL), _f32),
                   jax.ShapeDtypeStruct((1, NPAD), _f32)),
    )(Wt, hT, degp)


def _mmt_body(wt_ref, h_ref, dis_ref, o_ref):
    o_ref[...] = (jnp.dot(wt_ref[...], h_ref[...],
                          preferred_element_type=_f32)
                  * dis_ref[...][:, :NREAL])


def _mmt(Wt, hT, dis2d):
    return pl.pallas_call(
        _mmt_body,
        out_shape=jax.ShapeDtypeStruct((D, NREAL), _f32),
    )(Wt, hT, dis2d)


@functools.partial(
    pl.kernel,
    out_type=(
        jax.ShapeDtypeStruct((NW, NPAD), _f32),
        jax.ShapeDtypeStruct((E,), _i32),
    ),
    mesh=_MESH,
    compiler_params=_PARAMS,
    scratch_types=(
        pltpu.VMEM((NPAD,), _f32),
        pltpu.VMEM((EPW + L,), _i32),
        pltpu.VMEM((EPW + L,), _i32),
        pltpu.VMEM((EPW + L,), _f32),
    ),
)
def _phase_a(src_ref, dst_ref, ea_ref, degp_ref, opk_ref,
             degacc, csrc, cdst, cw):
    wid = lax.axis_index("s") * 2 + lax.axis_index("c")
    lane = lax.iota(_i32, L)
    zf = jnp.zeros((L,), _f32)

    @plsc.parallel_loop(0, NPAD // L, step=1, unroll=8)
    def _(i):
        degacc[pl.ds(i * L, L)] = zf

    base = wid * EPW
    pltpu.sync_copy(src_ref.at[pl.ds(base, EPW)], csrc.at[pl.ds(0, EPW)])
    pltpu.sync_copy(dst_ref.at[pl.ds(base, EPW)], cdst.at[pl.ds(0, EPW)])
    pltpu.sync_copy(ea_ref.at[pl.ds(base, EPW)], cw.at[pl.ds(0, EPW)])

    def edge_step(i, _):
        sl = pl.ds(i * L, L)
        srcv = csrc[sl]
        dstv = cdst[sl]
        wv = cw[sl]
        valid = (i * L + lane) < EPW
        dstv = jnp.where(valid, dstv, 0)
        pdst = dstv + 8 * (dstv >= HALF).astype(_i32)
        psrc = srcv + 8 * (srcv >= HALF).astype(_i32)
        csrc[sl] = psrc | (pdst << 16)
        plsc.addupdate_scatter(degacc, [pdst], wv, mask=valid)
        return 0
    lax.fori_loop(0, (EPW + L - 1) // L, edge_step, 0)
    pltpu.sync_copy(csrc.at[pl.ds(0, EPW)], opk_ref.at[pl.ds(base, EPW)])

    sbase = wid * SPW

    def self_step(i, _):
        node = sbase + i * L + lane
        valid = ((i * L + lane) < SPW) & (node < N)
        node = jnp.where(valid, node, 0)
        pnode = node + 8 * (node >= HALF).astype(_i32)
        ones = jnp.where(valid, 1.0, 0.0)
        plsc.addupdate_scatter(degacc, [pnode], ones, mask=valid)
        return 0
    lax.fori_loop(0, (SPW + L - 1) // L, self_step, 0)

    pltpu.sync_copy(degacc, degp_ref.at[wid])


@functools.partial(
    pl.kernel,
    out_type=jax.ShapeDtypeStruct((D, NREAL), _f32),
    mesh=_MESH,
    compiler_params=_PARAMS,
    scratch_types=(
        pltpu.VMEM((RPP, NREAL), _f32),
        pltpu.VMEM((RPP, NREAL), _f32),
        pltpu.VMEM((NPAD,), _f32),
        pltpu.VMEM((D,), _f32),
        pltpu.VMEM((CH,), _i32),
        pltpu.VMEM((CH,), _f32),
        pltpu.VMEM((CH,), _i32),
        pltpu.VMEM((CH,), _f32),
        pltpu.SemaphoreType.DMA,
        pltpu.SemaphoreType.DMA,
    ),
)
def _phase_b(y_ref, dis_ref, b_ref, pk_ref, ea_ref, out_ref,
             myh, acc, disv, bv, cpk, cwb, cpk2, cwb2, semA, semB):
    wid = lax.axis_index("s") * 2 + lax.axis_index("c")
    lane = lax.iota(_i32, L)
    zf = jnp.zeros((L,), _f32)

    pltpu.sync_copy(dis_ref, disv)
    pltpu.sync_copy(b_ref, bv)

    for p in range(2):
        row0 = wid * RPT + p * RPP
        pltpu.sync_copy(y_ref.at[pl.ds(row0, RPP)], myh)

        @plsc.parallel_loop(0, NSELF, step=1, unroll=8)
        def _(i):
            sl = pl.ds(i * L, L)
            for r in range(RPP):
                acc[r, sl] = zf

        def process(pkb, wb):
            @plsc.parallel_loop(0, NV, step=1, unroll=25)
            def _(i):
                sl = pl.ds(i * L, L)
                pk = pkb[sl]
                psrc = pk & 0xFFFF
                pdst = lax.shift_right_logical(pk, 16)
                wv = wb[sl]
                vals = [plsc.load_gather(myh, [jnp.full((L,), r, _i32), psrc])
                        * wv for r in range(RPP)]
                for r in range(RPP):
                    plsc.addupdate_scatter(
                        acc, [jnp.full((L,), r, _i32), pdst], vals[r])

        def start(jc, pkb, wb, sem):
            pltpu.async_copy(pk_ref.at[pl.ds(jc * CH, CH)], pkb, sem)
            pltpu.async_copy(ea_ref.at[pl.ds(jc * CH, CH)], wb, sem)

        def drain(pkb, wb, sem):
            pltpu.make_async_copy(pk_ref.at[pl.ds(0, CH)], pkb, sem).wait()
            pltpu.make_async_copy(ea_ref.at[pl.ds(0, CH)], wb, sem).wait()

        start(0, cpk, cwb, semA)

        def pair(jp, _):
            a = 2 * jp
            start(a + 1, cpk2, cwb2, semB)
            drain(cpk, cwb, semA)
            process(cpk, cwb)

            @pl.when(a + 2 < NCHK)
            def _():
                start(a + 2, cpk, cwb, semA)

            drain(cpk2, cwb2, semB)
            process(cpk2, cwb2)
            return 0
        lax.fori_loop(0, NCHK // 2, pair, 0)

        for r in range(RPP):
            brow = plsc.load_gather(bv, [jnp.full((L,), row0 + r, _i32)])

            @plsc.parallel_loop(0, NSELF, step=1, unroll=8)
            def _(i):
                sl = pl.ds(i * L, L)
                o = (acc[r, sl] + myh[r, sl]) * disv[sl] + brow
                acc[r, sl] = jnp.where(o > 0, o, NEG * o)
            pltpu.sync_copy(acc.at[r], out_ref.at[row0 + r])


def kernel(x, edge_index, edge_attr, W1, b1, W2, b2):
    src = edge_index[0]
    dst = edge_index[1]
    degp, pk = _phase_a(src, dst, edge_attr)
    zpad = jnp.zeros((8, D), _f32)
    xp = jnp.concatenate([x[:HALF], zpad, x[HALF:], zpad])
    y1, dis2d = _mmt1(W1.T, xp, degp)
    dis = dis2d.reshape(NPAD)
    h1 = _phase_b(y1, dis, b1, pk, edge_attr)
    y2 = _mmt(W2.T, h1, dis2d)
    h2 = _phase_b(y2, dis, b2, pk, edge_attr)
    h = h2.T
    return jnp.concatenate([h[:HALF], h[PHALF:PHALF + HALF]])
